# baseline (speedup 1.0000x reference)
import jax
import jax.numpy as jnp
from jax import lax
from jax.experimental import pallas as pl
from jax.experimental.pallas import tpu as pltpu

ROWS = 128
COLS = 32768
DROP = 512
KEEP = COLS - DROP
RB = 32
NCHUNK = ROWS // RB


def _tc_body(tok_hbm, out_ref, bufs, sems):
    copies = []
    for c in range(NCHUNK):
        cp = pltpu.make_async_copy(
            tok_hbm.at[pl.ds(c * RB, RB), :], bufs.at[c], sems.at[c])
        copies.append(cp)
    copies[0].start()
    cols = lax.broadcasted_iota(jnp.int32, (RB, COLS), 1)
    m = cols >= DROP
    for c in range(NCHUNK):
        copies[c].wait()
        if c + 1 < NCHUNK:
            copies[c + 1].start()
        x = jnp.where(m, bufs[c], 0.0)
        out_ref[pl.ds(c * RB, RB), :] = (
            jnp.sum(x, axis=1, keepdims=True) * (1.0 / KEEP))


_tc_mean = pl.pallas_call(
    _tc_body,
    in_specs=[pl.BlockSpec(memory_space=pl.ANY)],
    out_specs=pl.BlockSpec(memory_space=pltpu.MemorySpace.VMEM),
    out_shape=jax.ShapeDtypeStruct((ROWS, 1), jnp.float32),
    scratch_shapes=[
        pltpu.VMEM((NCHUNK, RB, COLS), jnp.float32),
        pltpu.SemaphoreType.DMA((NCHUNK,)),
    ],
)


def kernel(tokens):
    return _tc_mean(tokens)


# speedup vs baseline: 1.2881x; 1.2881x over previous
import jax
import jax.numpy as jnp
from jax import lax
from jax.experimental import pallas as pl
from jax.experimental.pallas import tpu as pltpu

ROWS = 128
COLS = 32768
DROP = 512
KEEP = COLS - DROP


def _tc_body(tok_hbm, out_ref, buf, sem):
    cp = pltpu.make_async_copy(tok_hbm, buf, sem)
    cp.start()
    cp.wait()
    cols = lax.broadcasted_iota(jnp.int32, (ROWS, COLS), 1)
    x = jnp.where(cols >= DROP, buf[...], 0.0)
    out_ref[...] = jnp.sum(x, axis=1, keepdims=True) * (1.0 / KEEP)


_tc_mean = pl.pallas_call(
    _tc_body,
    in_specs=[pl.BlockSpec(memory_space=pl.ANY)],
    out_specs=pl.BlockSpec(memory_space=pltpu.MemorySpace.VMEM),
    out_shape=jax.ShapeDtypeStruct((ROWS, 1), jnp.float32),
    scratch_shapes=[
        pltpu.VMEM((ROWS, COLS), jnp.float32),
        pltpu.SemaphoreType.DMA,
    ],
)


def kernel(tokens):
    return _tc_mean(tokens)


# speedup vs baseline: 1.2971x; 1.0070x over previous
import jax
import jax.numpy as jnp
from jax.experimental import pallas as pl
from jax.experimental.pallas import tpu as pltpu

ROWS = 128
COLS = 32768
DROP = 512
KEEP = COLS - DROP


def _tc_body(tok_hbm, out_ref, buf, sem):
    cp = pltpu.make_async_copy(
        tok_hbm.at[:, pl.ds(DROP, KEEP)], buf, sem)
    cp.start()
    cp.wait()
    out_ref[...] = jnp.sum(buf[...], axis=1, keepdims=True) * (1.0 / KEEP)


_tc_mean = pl.pallas_call(
    _tc_body,
    in_specs=[pl.BlockSpec(memory_space=pl.ANY)],
    out_specs=pl.BlockSpec(memory_space=pltpu.MemorySpace.VMEM),
    out_shape=jax.ShapeDtypeStruct((ROWS, 1), jnp.float32),
    scratch_shapes=[
        pltpu.VMEM((ROWS, KEEP), jnp.float32),
        pltpu.SemaphoreType.DMA,
    ],
)


def kernel(tokens):
    return _tc_mean(tokens)
